# Initial kernel scaffold; baseline (speedup 1.0000x reference)
import jax
import jax.numpy as jnp
from jax.experimental import pallas as pl


def kernel(x, edge_index, edge_attr, Wl0, Wr0, We0, att0, b0, Wl1, Wr1, We1, att1, b1, Wl2, Wr2, We2, att2, b2):
    raise NotImplementedError("write your pallas kernel here")


# speedup vs baseline: 10.9246x; 10.9246x over previous
import functools

import jax
import jax.numpy as jnp
from jax import lax
from jax.experimental import pallas as pl
from jax.experimental.pallas import tpu as pltpu
from jax.experimental.pallas import tpu_sc as plsc

N = 10000
NP = 10240
E = 320000
L = 16
NC, NS = 2, 16
NW = NC * NS
EP = 327680
EPW = EP // NW
RPT = NP // NS
ZC = 64


def _make_sc_edge(heads, d, C, ZC):
  rw = d + 16
  NCHUNK = EPW // C
  mesh = plsc.VectorSubcoreMesh(core_axis_name="c", subcore_axis_name="s")

  @functools.partial(
      pl.kernel,
      out_type=jax.ShapeDtypeStruct((NC, NP, rw), jnp.float32),
      mesh=mesh,
      compiler_params=pltpu.CompilerParams(needs_layout_passes=False,
                                           use_tc_tiling_on_sc=False),
      scratch_types=[
          pltpu.VMEM((C,), jnp.int32),
          pltpu.VMEM((C,), jnp.int32),
          pltpu.VMEM((C, d), jnp.float32),
          pltpu.VMEM((C, d), jnp.float32),
          pltpu.VMEM((C, d), jnp.float32),
          pltpu.VMEM((C, rw), jnp.float32),
          pltpu.VMEM((d,), jnp.float32),
          pltpu.VMEM((ZC, rw), jnp.float32),
          pltpu.VMEM_SHARED((NP, rw), jnp.float32),
          pltpu.SemaphoreType.DMA,
          pltpu.SemaphoreType.DMA,
      ],
  )
  def sc_kernel(src_hbm, dst_hbm, xl_hbm, xr_hbm, ea_hbm, att_hbm, zeros_hbm,
                out_hbm, srcv, dstv, xlr, xrr, ear, wout, attv, zbuf, acc,
                sem1, sem2):
    cid = lax.axis_index("c")
    sid = lax.axis_index("s")
    wid = cid * NS + sid

    pltpu.sync_copy(zeros_hbm, zbuf)
    for t in range(RPT // ZC):
      pltpu.sync_copy(zbuf, acc.at[pl.ds(sid * RPT + t * ZC, ZC)])
    pltpu.sync_copy(att_hbm, attv)
    plsc.subcore_barrier()

    iot = lax.iota(jnp.int32, L)
    zv = jnp.zeros((L,), jnp.float32)
    for e in range(C):
      wout[e, pl.ds(d, L)] = zv

    def chunk_body(j, carry):
      base = wid * EPW + j * C
      pltpu.sync_copy(src_hbm.at[pl.ds(base, C)], srcv)
      pltpu.sync_copy(dst_hbm.at[pl.ds(base, C)], dstv)
      cp1 = pltpu.async_copy(xl_hbm.at[srcv], xlr, sem1)
      cp2 = pltpu.async_copy(xr_hbm.at[dstv], xrr, sem2)
      pltpu.sync_copy(ea_hbm.at[pl.ds(base, C)], ear)
      cp1.wait()
      cp2.wait()

      def group_body(g, carry2):
        eids = g * L + iot
        if heads > 1:
          for h in range(heads):
            xls = []
            lg = zv
            atth = attv[pl.ds(h * L, L)]
            for cc in range(L):
              ci = jnp.full((L,), h * L + cc, jnp.int32)
              xlc = plsc.load_gather(xlr, [eids, ci])
              xrc = plsc.load_gather(xrr, [eids, ci])
              eac = plsc.load_gather(ear, [eids, ci])
              s = xlc + xrc + eac
              m = jnp.maximum(s, 0.2 * s)
              lg = lg + m * atth[cc]
              xls.append(xlc)
            w = jnp.exp(lg)
            for cc in range(L):
              ci = jnp.full((L,), h * L + cc, jnp.int32)
              plsc.store_scatter(wout, [eids, ci], w * xls[cc])
            plsc.store_scatter(wout, [eids, jnp.full((L,), d + h, jnp.int32)], w)
        else:
          def logit_blk(k, lg_c):
            attk = attv[pl.ds(k * L, L)]
            for cc in range(L):
              ci = jnp.full((L,), k * L + cc, jnp.int32)
              xlc = plsc.load_gather(xlr, [eids, ci])
              xrc = plsc.load_gather(xrr, [eids, ci])
              eac = plsc.load_gather(ear, [eids, ci])
              s = xlc + xrc + eac
              m = jnp.maximum(s, 0.2 * s)
              lg_c = lg_c + m * attk[cc]
            return lg_c

          w = jnp.exp(lax.fori_loop(0, d // L, logit_blk, zv))

          def out_blk(k, carry3):
            for cc in range(L):
              ci = jnp.full((L,), k * L + cc, jnp.int32)
              xlc = plsc.load_gather(xlr, [eids, ci])
              plsc.store_scatter(wout, [eids, ci], w * xlc)
            return carry3

          lax.fori_loop(0, d // L, out_blk, 0)
          plsc.store_scatter(wout, [eids, jnp.full((L,), d, jnp.int32)], w)
        return carry2

      lax.fori_loop(0, C // L, group_body, 0)
      pltpu.sync_copy(wout, acc.at[dstv], add=True)
      return carry

    lax.fori_loop(0, NCHUNK, chunk_body, 0)
    plsc.subcore_barrier()

    for t in range(RPT // ZC):
      r0 = sid * RPT + t * ZC
      pltpu.sync_copy(acc.at[pl.ds(r0, ZC)], zbuf)
      pltpu.sync_copy(zbuf, out_hbm.at[cid, pl.ds(r0, ZC)])

  return sc_kernel


_sc_edge6 = _make_sc_edge(6, 96, 80, 64)
_sc_edge1 = _make_sc_edge(1, 128, 64, 32)


def _dotT(x, w):
  return lax.dot_general(x, w, (((1,), (1,)), ((), ())),
                         preferred_element_type=jnp.float32)


def _proj_body(x_ref, wl_ref, wr_ref, xl_ref, xr_ref):
  x = x_ref[...]
  xl_ref[...] = _dotT(x, wl_ref[...])
  xr_ref[...] = _dotT(x, wr_ref[...])


def _proj(x, wl, wr, rb=1000):
  n, k = x.shape
  m = wl.shape[0]
  return pl.pallas_call(
      _proj_body,
      grid=(n // rb,),
      in_specs=[
          pl.BlockSpec((rb, k), lambda i: (i, 0)),
          pl.BlockSpec((m, k), lambda i: (0, 0)),
          pl.BlockSpec((m, k), lambda i: (0, 0)),
      ],
      out_specs=[pl.BlockSpec((rb, m), lambda i: (i, 0))] * 2,
      out_shape=[jax.ShapeDtypeStruct((n, m), jnp.float32)] * 2,
  )(x, wl, wr)


def _ea_body(ea_ref, w0_ref, w1_ref, w2_ref, o0_ref, o1_ref, o2_ref):
  ea = ea_ref[...]
  o0_ref[...] = _dotT(ea, w0_ref[...])
  o1_ref[...] = _dotT(ea, w1_ref[...])
  o2_ref[...] = _dotT(ea, w2_ref[...])


def _ea_proj(ea, w0, w1, w2, rb=4096):
  e, k = ea.shape
  ms = [w0.shape[0], w1.shape[0], w2.shape[0]]
  return pl.pallas_call(
      _ea_body,
      grid=(e // rb,),
      in_specs=[pl.BlockSpec((rb, k), lambda i: (i, 0))] +
               [pl.BlockSpec((m, k), lambda i: (0, 0)) for m in ms],
      out_specs=[pl.BlockSpec((rb, m), lambda i: (i, 0)) for m in ms],
      out_shape=[jax.ShapeDtypeStruct((e, m), jnp.float32) for m in ms],
  )(ea, w0, w1, w2)


def _norm(p, heads, d):
  s = p[0] + p[1]
  parts = []
  for h in range(heads):
    den = s[:, d + h:d + h + 1] + 1e-16
    parts.append(s[:, h * L:(h + 1) * L] / den)
  if heads > 1:
    return jnp.concatenate(parts, axis=1)
  return parts[0]


def _norm1(p, d):
  s = p[0] + p[1]
  return s[:, :d] / (s[:, d:d + 1] + 1e-16)


def _norm_proj_body(heads, d, p_ref, b_ref, wl_ref, wr_ref, xl_ref, xr_ref):
  if heads > 1:
    out = _norm(p_ref[...], heads, d)
  else:
    out = _norm1(p_ref[...], d)
  h = out + b_ref[...]
  h = jnp.where(h > 0, h, jnp.exp(h) - 1.0)
  xl_ref[...] = _dotT(h, wl_ref[...])
  xr_ref[...] = _dotT(h, wr_ref[...])


def _norm_proj(p, b, wl, wr, heads, d, rb=1024):
  m = wl.shape[0]
  rw = p.shape[2]
  return pl.pallas_call(
      functools.partial(_norm_proj_body, heads, d),
      grid=(NP // rb,),
      in_specs=[
          pl.BlockSpec((2, rb, rw), lambda i: (0, i, 0)),
          pl.BlockSpec((1, d), lambda i: (0, 0)),
          pl.BlockSpec((m, d), lambda i: (0, 0)),
          pl.BlockSpec((m, d), lambda i: (0, 0)),
      ],
      out_specs=[pl.BlockSpec((rb, m), lambda i: (i, 0))] * 2,
      out_shape=[jax.ShapeDtypeStruct((NP, m), jnp.float32)] * 2,
  )(p, b.reshape(1, d), wl, wr)


def _final_body(d, p_ref, b_ref, o_ref):
  o_ref[...] = _norm1(p_ref[...], d) + b_ref[...]


def _final(p, b, d, rb=1024):
  rw = p.shape[2]
  return pl.pallas_call(
      functools.partial(_final_body, d),
      grid=(NP // rb,),
      in_specs=[
          pl.BlockSpec((2, rb, rw), lambda i: (0, i, 0)),
          pl.BlockSpec((1, d), lambda i: (0, 0)),
      ],
      out_specs=pl.BlockSpec((rb, d), lambda i: (i, 0)),
      out_shape=jax.ShapeDtypeStruct((NP, d), jnp.float32),
  )(p, b.reshape(1, d))


def kernel(x, edge_index, edge_attr, Wl0, Wr0, We0, att0, b0,
           Wl1, Wr1, We1, att1, b1, Wl2, Wr2, We2, att2, b2):
  src = jnp.concatenate([edge_index[0], jnp.zeros((EP - E,), jnp.int32)])
  dst = jnp.concatenate([edge_index[1],
                         jnp.full((EP - E,), NP - 1, jnp.int32)])
  edge_attr = jnp.concatenate(
      [edge_attr, jnp.zeros((EP - E, edge_attr.shape[1]), jnp.float32)])
  x = jnp.concatenate([x, jnp.zeros((NP - N, x.shape[1]), jnp.float32)])
  z112 = jnp.zeros((64, 112), jnp.float32)
  z144 = jnp.zeros((32, 144), jnp.float32)

  ea0, ea1, ea2 = _ea_proj(edge_attr, We0, We1, We2)

  xl0, xr0 = _proj(x, Wl0, Wr0, rb=1024)
  p0 = _sc_edge6(src, dst, xl0, xr0, ea0, att0.reshape(96), z112)
  xl1, xr1 = _norm_proj(p0, b0, Wl1, Wr1, 6, 96)
  p1 = _sc_edge6(src, dst, xl1, xr1, ea1, att1.reshape(96), z112)
  xl2, xr2 = _norm_proj(p1, b1, Wl2, Wr2, 6, 96)
  p2 = _sc_edge1(src, dst, xl2, xr2, ea2, att2.reshape(128), z144)
  return _final(p2, b2, 128)[:N]
